# baseline (speedup 1.0000x reference)
import jax
import jax.numpy as jnp
from jax import lax
from jax.experimental import pallas as pl
from jax.experimental.pallas import tpu as pltpu
from jax.experimental.pallas import tpu_sc as plsc

BATCH = 4096
HIST = 50
EMB_DIM = 64
PAD_DIM = 128

NC = 2
NS = 16
NW = NC * NS

BW = BATCH // NW
HC = 2
NCH = HIST // HC
LANES = 16
DBLK = EMB_DIM // LANES
BBLK = BW // LANES


def _body(xt_hbm, table_hbm, out_hbm, idx_v, gbuf, tbuf, sem_g, sem_o):
    wid = lax.axis_index("s") * NC + lax.axis_index("c")
    b0 = wid * BW

    pltpu.sync_copy(xt_hbm.at[:, pl.ds(b0, BW)], idx_v)

    lane = jnp.arange(LANES, dtype=jnp.int32)
    diags = [(lane + k) & (LANES - 1) for k in range(LANES)]

    def gather_copy(g, p, hh):
        return pltpu.make_async_copy(
            table_hbm.at[idx_v.at[g * HC + hh]],
            gbuf.at[p, hh],
            sem_g,
        )

    def gather_start(g, p):
        for hh in range(HC):
            gather_copy(g, p, hh).start()

    def gather_wait(g, p):
        for hh in range(HC):
            gather_copy(g, p, hh).wait()

    def store_copy(g, p):
        return pltpu.make_async_copy(
            tbuf.at[p],
            out_hbm.at[pl.ds(g * HC, HC), :, pl.ds(b0, BW)],
            sem_o,
        )

    def transpose(p):
        for hh in range(HC):
            rows = gbuf.at[p, hh]
            tp = tbuf.at[p, hh]

            @plsc.parallel_loop(0, DBLK * BBLK, unroll=2)
            def _(bi):
                d0 = (bi % DBLK) * LANES
                bb = (bi // DBLK) * LANES
                row = bb + lane
                for k in range(LANES):
                    dcol = d0 + diags[k]
                    v = plsc.load_gather(rows, [row, dcol])
                    plsc.store_scatter(tp, [dcol, row], v)

    gather_start(0, 0)

    def chunk(g, carry):
        p = lax.rem(g, 2)
        gather_wait(g, p)

        @pl.when(g + 1 < NCH)
        def _():
            gather_start(g + 1, 1 - p)

        @pl.when(g >= 2)
        def _():
            store_copy(g - 2, p).wait()

        transpose(p)
        store_copy(g, p).start()
        return carry

    lax.fori_loop(0, NCH, chunk, 0)
    store_copy(NCH - 2, (NCH - 2) % 2).wait()
    store_copy(NCH - 1, (NCH - 1) % 2).wait()


@jax.jit
def _gather(xt, table_p):
    run = pl.kernel(
        _body,
        out_type=jax.ShapeDtypeStruct((HIST, EMB_DIM, BATCH), jnp.float32),
        mesh=plsc.VectorSubcoreMesh(core_axis_name="c", subcore_axis_name="s"),
        compiler_params=pltpu.CompilerParams(
            use_tc_tiling_on_sc=True, needs_layout_passes=False
        ),
        scratch_types=[
            pltpu.VMEM((HIST, BW), jnp.int32),
            pltpu.VMEM((2, HC, BW, PAD_DIM), jnp.float32),
            pltpu.VMEM((2, HC, EMB_DIM, BW), jnp.float32),
            pltpu.SemaphoreType.DMA,
            pltpu.SemaphoreType.DMA,
        ],
    )
    return run(xt, table_p)


def kernel(x, table):
    xt = x.astype(jnp.int32).T
    table_p = jnp.concatenate(
        [table, jnp.zeros((table.shape[0], PAD_DIM - EMB_DIM), table.dtype)], 1
    )
    out_t = _gather(xt, table_p)
    return jnp.transpose(out_t, (2, 0, 1))


# speedup vs baseline: 1.0004x; 1.0004x over previous
import jax
import jax.numpy as jnp
from jax import lax
from jax.experimental import pallas as pl
from jax.experimental.pallas import tpu as pltpu
from jax.experimental.pallas import tpu_sc as plsc

BATCH = 4096
HIST = 50
EMB_DIM = 64
PAD_DIM = 128

NC = 2
NS = 16
NW = NC * NS

BW = BATCH // NW
HC = 2
NCH = HIST // HC
LANES = 16
DBLK = EMB_DIM // LANES
BBLK = BW // LANES


def _body(xt_hbm, table_hbm, out_hbm, idx_v, gbuf, tbuf, sem_g, sem_o):
    wid = lax.axis_index("s") * NC + lax.axis_index("c")
    b0 = wid * BW

    pltpu.sync_copy(xt_hbm.at[:, pl.ds(b0, BW)], idx_v)

    lane = jnp.arange(LANES, dtype=jnp.int32)
    diags = [(lane + k) & (LANES - 1) for k in range(LANES)]

    def gather_copy(g, p, hh):
        return pltpu.make_async_copy(
            table_hbm.at[idx_v.at[g * HC + hh]],
            gbuf.at[p, hh],
            sem_g,
        )

    def gather_start(g, p):
        for hh in range(HC):
            gather_copy(g, p, hh).start()

    def gather_wait(g, p):
        for hh in range(HC):
            gather_copy(g, p, hh).wait()

    def store_copy(g, p):
        return pltpu.make_async_copy(
            tbuf.at[p],
            out_hbm.at[pl.ds(g * HC, HC), :, pl.ds(b0, BW)],
            sem_o,
        )

    def transpose(p):
        for hh in range(HC):
            rows = gbuf.at[p, hh]
            tp = tbuf.at[p, hh]

            @plsc.parallel_loop(0, DBLK * BBLK, unroll=2)
            def _(bi):
                d0 = (bi % DBLK) * LANES
                bb = (bi // DBLK) * LANES
                row = bb + lane
                for k in range(LANES):
                    dcol = d0 + diags[k]
                    v = plsc.load_gather(rows, [row, dcol])
                    plsc.store_scatter(tp, [dcol, row], v)

    gather_start(0, 0)

    def chunk(g, carry):
        p = lax.rem(g, 2)
        gather_wait(g, p)

        @pl.when(g + 1 < NCH)
        def _():
            gather_start(g + 1, 1 - p)

        @pl.when(g >= 2)
        def _():
            store_copy(g - 2, p).wait()

        transpose(p)
        store_copy(g, p).start()
        return carry

    lax.fori_loop(0, NCH, chunk, 0)
    store_copy(NCH - 2, (NCH - 2) % 2).wait()
    store_copy(NCH - 1, (NCH - 1) % 2).wait()


@jax.jit
def _gather(xt, table_p):
    run = pl.kernel(
        _body,
        out_type=jax.ShapeDtypeStruct((HIST, EMB_DIM, BATCH), jnp.float32),
        mesh=plsc.VectorSubcoreMesh(core_axis_name="c", subcore_axis_name="s"),
        compiler_params=pltpu.CompilerParams(
            use_tc_tiling_on_sc=True, needs_layout_passes=False
        ),
        scratch_types=[
            pltpu.VMEM((HIST, BW), jnp.int32),
            pltpu.VMEM((2, HC, BW, PAD_DIM), jnp.float32),
            pltpu.VMEM((2, HC, EMB_DIM, BW), jnp.float32),
            pltpu.SemaphoreType.DMA,
            pltpu.SemaphoreType.DMA,
        ],
    )
    return run(xt, table_p)


def kernel(x, table):
    xt = x.astype(jnp.int32).T
    table_p = jnp.pad(table, ((0, 0), (0, PAD_DIM - EMB_DIM)))
    out_t = _gather(xt, table_p)
    return jnp.transpose(out_t, (2, 0, 1))


# speedup vs baseline: 1.0470x; 1.0465x over previous
import jax
import jax.numpy as jnp
from jax import lax
from jax.experimental import pallas as pl
from jax.experimental.pallas import tpu as pltpu
from jax.experimental.pallas import tpu_sc as plsc

BATCH = 4096
HIST = 50
EMB_DIM = 64
PAD_DIM = 128

NC = 2
NS = 16
NW = NC * NS

BW = BATCH // NW
HC = 2
NCH = HIST // HC
LANES = 16
DBLK = EMB_DIM // LANES
BBLK = BW // LANES


def _body(xt_hbm, table_hbm, out_hbm, idx_v, gbuf, tbuf, sem_g, sem_o):
    wid = lax.axis_index("s") * NC + lax.axis_index("c")
    b0 = wid * BW

    pltpu.sync_copy(xt_hbm.at[:, pl.ds(b0, BW)], idx_v)

    lane = jnp.arange(LANES, dtype=jnp.int32)
    diags = [(lane + k) & (LANES - 1) for k in range(LANES)]

    def gather_copy(g, p, hh):
        return pltpu.make_async_copy(
            table_hbm.at[idx_v.at[g * HC + hh]],
            gbuf.at[p, hh],
            sem_g,
        )

    def gather_start(g, p):
        for hh in range(HC):
            gather_copy(g, p, hh).start()

    def gather_wait(g, p):
        for hh in range(HC):
            gather_copy(g, p, hh).wait()

    def store_copy(g, p):
        return pltpu.make_async_copy(
            tbuf.at[p],
            out_hbm.at[pl.ds(g * HC, HC), :, pl.ds(b0, BW)],
            sem_o,
        )

    def transpose(p):
        for hh in range(HC):
            rows = gbuf.at[p, hh]
            tp = tbuf.at[p, hh]

            @plsc.parallel_loop(0, DBLK * BBLK, unroll=2)
            def _(bi):
                d0 = (bi % DBLK) * LANES
                bb = (bi // DBLK) * LANES
                row = bb + lane
                for k in range(LANES):
                    dcol = d0 + diags[k]
                    v = plsc.load_gather(rows, [row, dcol])
                    plsc.store_scatter(tp, [dcol, row], v)

    gather_start(0, 0)

    def chunk(g, carry):
        p = lax.rem(g, 2)

        @pl.when(g + 1 < NCH)
        def _():
            gather_start(g + 1, 1 - p)

        gather_wait(g, p)

        @pl.when(g >= 2)
        def _():
            store_copy(g - 2, p).wait()

        transpose(p)
        store_copy(g, p).start()
        return carry

    lax.fori_loop(0, NCH, chunk, 0)
    store_copy(NCH - 2, (NCH - 2) % 2).wait()
    store_copy(NCH - 1, (NCH - 1) % 2).wait()


@jax.jit
def _gather(xt, table_p):
    run = pl.kernel(
        _body,
        out_type=jax.ShapeDtypeStruct((HIST, EMB_DIM, BATCH), jnp.float32),
        mesh=plsc.VectorSubcoreMesh(core_axis_name="c", subcore_axis_name="s"),
        compiler_params=pltpu.CompilerParams(
            use_tc_tiling_on_sc=True, needs_layout_passes=False
        ),
        scratch_types=[
            pltpu.VMEM((HIST, BW), jnp.int32),
            pltpu.VMEM((2, HC, BW, PAD_DIM), jnp.float32),
            pltpu.VMEM((2, HC, EMB_DIM, BW), jnp.float32),
            pltpu.SemaphoreType.DMA,
            pltpu.SemaphoreType.DMA,
        ],
    )
    return run(xt, table_p)


def kernel(x, table):
    xt = x.astype(jnp.int32).T
    table_p = jnp.pad(table, ((0, 0), (0, PAD_DIM - EMB_DIM)))
    out_t = _gather(xt, table_p)
    return jnp.transpose(out_t, (2, 0, 1))


# speedup vs baseline: 1.0669x; 1.0191x over previous
import jax
import jax.numpy as jnp
from jax import lax
from jax.experimental import pallas as pl
from jax.experimental.pallas import tpu as pltpu
from jax.experimental.pallas import tpu_sc as plsc

BATCH = 4096
HIST = 50
EMB_DIM = 64
PAD_DIM = 128

NC = 2
NS = 16
NW = NC * NS

BW = BATCH // NW
HC = 1
NCH = HIST // HC
LANES = 16
DBLK = EMB_DIM // LANES
BBLK = BW // LANES


def _body(xt_hbm, table_hbm, out_hbm, idx_v, gbuf, tbuf, sem_g, sem_o):
    wid = lax.axis_index("s") * NC + lax.axis_index("c")
    b0 = wid * BW

    pltpu.sync_copy(xt_hbm.at[:, pl.ds(b0, BW)], idx_v)

    lane = jnp.arange(LANES, dtype=jnp.int32)
    diags = [(lane + k) & (LANES - 1) for k in range(LANES)]

    def gather_copy(g, p, hh):
        return pltpu.make_async_copy(
            table_hbm.at[idx_v.at[g * HC + hh]],
            gbuf.at[p, hh],
            sem_g,
        )

    def gather_start(g, p):
        for hh in range(HC):
            gather_copy(g, p, hh).start()

    def gather_wait(g, p):
        for hh in range(HC):
            gather_copy(g, p, hh).wait()

    def store_copy(g, p):
        return pltpu.make_async_copy(
            tbuf.at[p],
            out_hbm.at[pl.ds(g * HC, HC), :, pl.ds(b0, BW)],
            sem_o,
        )

    def transpose(q, p):
        for hh in range(HC):
            rows = gbuf.at[q, hh]
            tp = tbuf.at[p, hh]

            @plsc.parallel_loop(0, DBLK * BBLK, unroll=2)
            def _(bi):
                d0 = (bi % DBLK) * LANES
                bb = (bi // DBLK) * LANES
                row = bb + lane
                for k in range(LANES):
                    dcol = d0 + diags[k]
                    v = plsc.load_gather(rows, [row, dcol])
                    plsc.store_scatter(tp, [dcol, row], v)

    gather_start(0, 0)
    gather_start(1, 1)

    def chunk(g, carry):
        q = lax.rem(g, 3)
        p = lax.rem(g, 2)

        @pl.when(g + 2 < NCH)
        def _():
            gather_start(g + 2, lax.rem(g + 2, 3))

        gather_wait(g, q)

        @pl.when(g >= 2)
        def _():
            store_copy(g - 2, p).wait()

        transpose(q, p)
        store_copy(g, p).start()
        return carry

    lax.fori_loop(0, NCH, chunk, 0)
    store_copy(NCH - 2, (NCH - 2) % 2).wait()
    store_copy(NCH - 1, (NCH - 1) % 2).wait()


@jax.jit
def _gather(xt, table_p):
    run = pl.kernel(
        _body,
        out_type=jax.ShapeDtypeStruct((HIST, EMB_DIM, BATCH), jnp.float32),
        mesh=plsc.VectorSubcoreMesh(core_axis_name="c", subcore_axis_name="s"),
        compiler_params=pltpu.CompilerParams(
            use_tc_tiling_on_sc=True, needs_layout_passes=False
        ),
        scratch_types=[
            pltpu.VMEM((HIST, BW), jnp.int32),
            pltpu.VMEM((3, HC, BW, PAD_DIM), jnp.float32),
            pltpu.VMEM((2, HC, EMB_DIM, BW), jnp.float32),
            pltpu.SemaphoreType.DMA,
            pltpu.SemaphoreType.DMA,
        ],
    )
    return run(xt, table_p)


def kernel(x, table):
    xt = x.astype(jnp.int32).T
    table_p = jnp.pad(table, ((0, 0), (0, PAD_DIM - EMB_DIM)))
    out_t = _gather(xt, table_p)
    return jnp.transpose(out_t, (2, 0, 1))
